# Initial kernel scaffold; baseline (speedup 1.0000x reference)
import functools

import jax
import jax.numpy as jnp
from jax import lax
from jax.experimental import pallas as pl
from jax.experimental.pallas import tpu as pltpu
from jax.experimental.pallas import tpu_sc as plsc

T = 2048
D = 1024
F = 2048
E = 8
K = 2

BLK = 128
NT = 39
P_CAP = NT * BLK

NC, NS = 2, 16
NW = NC * NS
TPW = T // NW
KC = 32
CH = 128
NCH = (K * T) // CH

_f32 = jnp.float32
_i32 = jnp.int32


def _router_body(x_ref, gw_ref, pos_ref, w0_ref, w1_ref, te_ref, tv_ref,
                 oh_scr, rank_scr):
    x = x_ref[...]
    logits = lax.dot_general(x, gw_ref[...], (((1,), (1,)), ((), ())),
                             preferred_element_type=_f32)
    m = jnp.max(logits, axis=1, keepdims=True)
    p = jnp.exp(logits - m)
    probs = p / jnp.sum(p, axis=1, keepdims=True)

    iota_e = lax.broadcasted_iota(_i32, (T, E), 1)
    v0 = jnp.max(probs, axis=1, keepdims=True)
    i0 = jnp.min(jnp.where(probs == v0, iota_e, E), axis=1, keepdims=True)
    probs2 = jnp.where(iota_e == i0, -1.0, probs)
    v1 = jnp.max(probs2, axis=1, keepdims=True)
    i1 = jnp.min(jnp.where(probs2 == v1, iota_e, E), axis=1, keepdims=True)
    s = v0 + v1
    w0_ref[...] = jnp.broadcast_to(v0 / s, (T, 128))
    w1_ref[...] = jnp.broadcast_to(v1 / s, (T, 128))

    e_pair = jnp.concatenate([i0, i1], axis=0)
    onehot = (jnp.broadcast_to(e_pair, (K * T, E))
              == lax.broadcasted_iota(_i32, (K * T, E), 1)).astype(_f32)
    oh_scr[...] = onehot

    tri = (lax.broadcasted_iota(_i32, (CH, CH), 0)
           >= lax.broadcasted_iota(_i32, (CH, CH), 1)).astype(_f32)

    def step(c, tot):
        off = pl.multiple_of(c * CH, CH)
        ob = oh_scr[pl.ds(off, CH), :]
        inc = lax.dot_general(tri, ob, (((1,), (0,)), ((), ())),
                              preferred_element_type=_f32)
        rank_scr[pl.ds(off, CH), :] = ob * (inc + jnp.broadcast_to(tot, (CH, E)))
        return tot + jnp.sum(ob, axis=0, keepdims=True)

    counts = lax.fori_loop(0, NCH, step, jnp.zeros((1, E), _f32))

    ci = counts.astype(_i32)
    pc = (((ci + (BLK - 1)) // BLK) * BLK).astype(_f32)
    su = (lax.broadcasted_iota(_i32, (E, E), 0)
          < lax.broadcasted_iota(_i32, (E, E), 1)).astype(_f32)
    off_e = lax.dot_general(pc, su, (((1,), (0,)), ((), ())))
    cum_incl = off_e + pc

    rank_incl = jnp.sum(rank_scr[...], axis=1, keepdims=True)
    slot_base = lax.dot_general(oh_scr[...], off_e,
                                (((1,), (1,)), ((), ())))
    pos_ref[...] = (rank_incl - 1.0 + slot_base).astype(_i32)

    starts = (lax.broadcasted_iota(_i32, (1, 128), 1) * BLK).astype(_f32)
    cnt = jnp.zeros((1, 128), _f32)
    for e in range(E):
        ce = jnp.broadcast_to(cum_incl[0:1, e:e + 1], (1, 128))
        cnt += (starts >= ce).astype(_f32)
    te_ref[...] = jnp.minimum(cnt, float(E - 1)).astype(_i32)
    total = jnp.broadcast_to(cum_incl[0:1, E - 1:E], (1, 128))
    tv_ref[...] = (starts < total).astype(_i32)


_router = pl.pallas_call(
    _router_body,
    out_shape=(
        jax.ShapeDtypeStruct((K * T, 1), _i32),
        jax.ShapeDtypeStruct((T, 128), _f32),
        jax.ShapeDtypeStruct((T, 128), _f32),
        jax.ShapeDtypeStruct((1, 128), _i32),
        jax.ShapeDtypeStruct((1, 128), _i32),
    ),
    scratch_shapes=[
        pltpu.VMEM((K * T, E), _f32),
        pltpu.VMEM((K * T, E), _f32),
    ],
)


def _dispatch_body(x_hbm, pos0_hbm, pos1_hbm, xs_hbm, idx0_v, idx1_v, rows_v, sem):
    wid = lax.axis_index("s") * NC + lax.axis_index("c")
    base = wid * TPW
    pltpu.sync_copy(pos0_hbm.at[pl.ds(base, TPW)], idx0_v)
    pltpu.sync_copy(pos1_hbm.at[pl.ds(base, TPW)], idx1_v)
    pltpu.sync_copy(x_hbm.at[pl.ds(base, TPW)], rows_v)
    pltpu.async_copy(rows_v, xs_hbm.at[idx0_v], sem).wait()
    pltpu.async_copy(rows_v, xs_hbm.at[idx1_v], sem).wait()


_dispatch = pl.kernel(
    _dispatch_body,
    out_type=jax.ShapeDtypeStruct((P_CAP, D), _f32),
    mesh=plsc.VectorSubcoreMesh(core_axis_name="c", subcore_axis_name="s",
                                num_cores=NC, num_subcores=NS),
    scratch_types=[
        pltpu.VMEM((TPW,), _i32),
        pltpu.VMEM((TPW,), _i32),
        pltpu.VMEM((TPW, D), _f32),
        pltpu.SemaphoreType.DMA,
    ],
)


def _ffn_body(te_ref, tv_ref, xs_ref, wg_ref, wu_ref, wd_ref, out_ref):
    i = pl.program_id(0)

    @pl.when(tv_ref[i] == 1)
    def _():
        x = xs_ref[...]
        g = lax.dot_general(x, wg_ref[0], (((1,), (1,)), ((), ())),
                            preferred_element_type=_f32)
        u = lax.dot_general(x, wu_ref[0], (((1,), (1,)), ((), ())),
                            preferred_element_type=_f32)
        h = g * u / (1.0 + jnp.exp(-g))
        out_ref[...] = lax.dot_general(h, wd_ref[0], (((1,), (1,)), ((), ())),
                                       preferred_element_type=_f32)


_ffn = pl.pallas_call(
    _ffn_body,
    grid_spec=pltpu.PrefetchScalarGridSpec(
        num_scalar_prefetch=2,
        grid=(NT,),
        in_specs=[
            pl.BlockSpec((BLK, D), lambda i, te, tv: (i, 0)),
            pl.BlockSpec((1, F, D), lambda i, te, tv: (te[i], 0, 0)),
            pl.BlockSpec((1, F, D), lambda i, te, tv: (te[i], 0, 0)),
            pl.BlockSpec((1, D, F), lambda i, te, tv: (te[i], 0, 0)),
        ],
        out_specs=pl.BlockSpec((BLK, D), lambda i, te, tv: (i, 0)),
    ),
    out_shape=jax.ShapeDtypeStruct((P_CAP, D), _f32),
)


def _combine_body(buf_hbm, pos0_hbm, pos1_hbm, w0_hbm, w1_hbm, out_hbm,
                  idx0_v, idx1_v, a_v, b_v, o_v, w0_v, w1_v, sem):
    wid = lax.axis_index("s") * NC + lax.axis_index("c")
    for c in range(TPW // KC):
        base = wid * TPW + c * KC
        pltpu.sync_copy(pos0_hbm.at[pl.ds(base, KC)], idx0_v)
        pltpu.sync_copy(pos1_hbm.at[pl.ds(base, KC)], idx1_v)
        pltpu.sync_copy(w0_hbm.at[pl.ds(base, KC)], w0_v)
        pltpu.sync_copy(w1_hbm.at[pl.ds(base, KC)], w1_v)
        pltpu.async_copy(buf_hbm.at[idx0_v], a_v, sem).wait()
        pltpu.async_copy(buf_hbm.at[idx1_v], b_v, sem).wait()

        def row(r, carry):
            wa = w0_v[r, pl.ds(0, 16)]
            wb = w1_v[r, pl.ds(0, 16)]

            def dchunk(j, inner):
                b0 = pl.multiple_of(j * 64, 64)
                for q in range(4):
                    sl = pl.ds(b0 + q * 16, 16)
                    o_v[r, sl] = wa * a_v[r, sl] + wb * b_v[r, sl]
                return inner

            lax.fori_loop(0, D // 64, dchunk, 0)
            return carry

        lax.fori_loop(0, KC, row, 0)
        pltpu.sync_copy(o_v, out_hbm.at[pl.ds(base, KC)])


_combine = pl.kernel(
    _combine_body,
    out_type=jax.ShapeDtypeStruct((T, D), _f32),
    mesh=plsc.VectorSubcoreMesh(core_axis_name="c", subcore_axis_name="s",
                                num_cores=NC, num_subcores=NS),
    scratch_types=[
        pltpu.VMEM((KC,), _i32),
        pltpu.VMEM((KC,), _i32),
        pltpu.VMEM((KC, D), _f32),
        pltpu.VMEM((KC, D), _f32),
        pltpu.VMEM((KC, D), _f32),
        pltpu.VMEM((KC, 128), _f32),
        pltpu.VMEM((KC, 128), _f32),
        pltpu.SemaphoreType.DMA,
    ],
)


def kernel(hidden_states, gate_w, gate_proj_w, up_proj_w, down_proj_w):
    pos, w0b, w1b, te2, tv2 = _router(hidden_states, gate_w)
    pos_flat = pos.reshape(K * T)
    pos0, pos1 = pos_flat[:T], pos_flat[T:]
    te, tv = te2[0, :NT], tv2[0, :NT]
    xs = _dispatch(hidden_states, pos0, pos1)
    buf = _ffn(te, tv, xs, gate_proj_w, up_proj_w, down_proj_w)
    return _combine(buf, pos0, pos1, w0b, w1b)


# speedup vs baseline: 1.3781x; 1.3781x over previous
import functools

import jax
import jax.numpy as jnp
from jax import lax
from jax.experimental import pallas as pl
from jax.experimental.pallas import tpu as pltpu
from jax.experimental.pallas import tpu_sc as plsc

T = 2048
D = 1024
F = 2048
E = 8
K = 2

BLK = 128
NT = 39
P_CAP = NT * BLK

NC, NS = 2, 16
NW = NC * NS
TPW = T // NW
KC = 32
CH = 128
NCH = (K * T) // CH

_f32 = jnp.float32
_i32 = jnp.int32


def _router_body(x_ref, gw_ref, pos_ref, w0_ref, w1_ref, te_ref, tv_ref,
                 oh_scr, rank_scr):
    x = x_ref[...]
    logits = lax.dot_general(x, gw_ref[...], (((1,), (1,)), ((), ())),
                             preferred_element_type=_f32)
    m = jnp.max(logits, axis=1, keepdims=True)
    p = jnp.exp(logits - m)
    probs = p / jnp.sum(p, axis=1, keepdims=True)

    iota_e = lax.broadcasted_iota(_i32, (T, E), 1)
    v0 = jnp.max(probs, axis=1, keepdims=True)
    i0 = jnp.min(jnp.where(probs == v0, iota_e, E), axis=1, keepdims=True)
    probs2 = jnp.where(iota_e == i0, -1.0, probs)
    v1 = jnp.max(probs2, axis=1, keepdims=True)
    i1 = jnp.min(jnp.where(probs2 == v1, iota_e, E), axis=1, keepdims=True)
    s = v0 + v1
    w0_ref[...] = jnp.broadcast_to(v0 / s, (T, 128))
    w1_ref[...] = jnp.broadcast_to(v1 / s, (T, 128))

    e_pair = jnp.concatenate([i0, i1], axis=0)
    onehot = (jnp.broadcast_to(e_pair, (K * T, E))
              == lax.broadcasted_iota(_i32, (K * T, E), 1)).astype(_f32)
    oh_scr[...] = onehot

    tri = (lax.broadcasted_iota(_i32, (CH, CH), 0)
           >= lax.broadcasted_iota(_i32, (CH, CH), 1)).astype(_f32)

    def step(c, tot):
        off = pl.multiple_of(c * CH, CH)
        ob = oh_scr[pl.ds(off, CH), :]
        inc = lax.dot_general(tri, ob, (((1,), (0,)), ((), ())),
                              preferred_element_type=_f32)
        rank_scr[pl.ds(off, CH), :] = ob * (inc + jnp.broadcast_to(tot, (CH, E)))
        return tot + jnp.sum(ob, axis=0, keepdims=True)

    counts = lax.fori_loop(0, NCH, step, jnp.zeros((1, E), _f32))

    ci = counts.astype(_i32)
    pc = (((ci + (BLK - 1)) // BLK) * BLK).astype(_f32)
    su = (lax.broadcasted_iota(_i32, (E, E), 0)
          < lax.broadcasted_iota(_i32, (E, E), 1)).astype(_f32)
    off_e = lax.dot_general(pc, su, (((1,), (0,)), ((), ())))
    cum_incl = off_e + pc

    rank_incl = jnp.sum(rank_scr[...], axis=1, keepdims=True)
    slot_base = lax.dot_general(oh_scr[...], off_e,
                                (((1,), (1,)), ((), ())))
    pos_ref[...] = (rank_incl - 1.0 + slot_base).astype(_i32)

    starts = (lax.broadcasted_iota(_i32, (1, 128), 1) * BLK).astype(_f32)
    cnt = jnp.zeros((1, 128), _f32)
    for e in range(E):
        ce = jnp.broadcast_to(cum_incl[0:1, e:e + 1], (1, 128))
        cnt += (starts >= ce).astype(_f32)
    te_ref[...] = jnp.minimum(cnt, float(E - 1)).astype(_i32)
    total = jnp.broadcast_to(cum_incl[0:1, E - 1:E], (1, 128))
    tv_ref[...] = (starts < total).astype(_i32)


_router = pl.pallas_call(
    _router_body,
    out_shape=(
        jax.ShapeDtypeStruct((K * T, 1), _i32),
        jax.ShapeDtypeStruct((T, 128), _f32),
        jax.ShapeDtypeStruct((T, 128), _f32),
        jax.ShapeDtypeStruct((1, 128), _i32),
        jax.ShapeDtypeStruct((1, 128), _i32),
    ),
    scratch_shapes=[
        pltpu.VMEM((K * T, E), _f32),
        pltpu.VMEM((K * T, E), _f32),
    ],
)


def _dispatch_body(x_hbm, pos0_hbm, pos1_hbm, xs_hbm, idx0_v, idx1_v, rows_v, sem):
    wid = lax.axis_index("s") * NC + lax.axis_index("c")
    base = wid * TPW
    pltpu.sync_copy(pos0_hbm.at[pl.ds(base, TPW)], idx0_v)
    pltpu.sync_copy(pos1_hbm.at[pl.ds(base, TPW)], idx1_v)
    pltpu.sync_copy(x_hbm.at[pl.ds(base, TPW)], rows_v)
    pltpu.async_copy(rows_v, xs_hbm.at[idx0_v], sem).wait()
    pltpu.async_copy(rows_v, xs_hbm.at[idx1_v], sem).wait()


@functools.cache
def _make_dispatch():
    return pl.kernel(
        _dispatch_body,
        out_type=jax.ShapeDtypeStruct((P_CAP, D), _f32),
        mesh=plsc.VectorSubcoreMesh(core_axis_name="c", subcore_axis_name="s",
                                    num_cores=NC, num_subcores=NS),
        scratch_types=[
            pltpu.VMEM((TPW,), _i32),
            pltpu.VMEM((TPW,), _i32),
            pltpu.VMEM((TPW, D), _f32),
            pltpu.SemaphoreType.DMA,
        ],
    )


def _ffn_body(te_ref, tv_ref, xs_ref, wg_ref, wu_ref, wd_ref, out_ref):
    i = pl.program_id(0)

    @pl.when(tv_ref[i] == 1)
    def _():
        x = xs_ref[...]
        g = lax.dot_general(x, wg_ref[0], (((1,), (1,)), ((), ())),
                            preferred_element_type=_f32)
        u = lax.dot_general(x, wu_ref[0], (((1,), (1,)), ((), ())),
                            preferred_element_type=_f32)
        h = g * u / (1.0 + jnp.exp(-g))
        out_ref[...] = lax.dot_general(h, wd_ref[0], (((1,), (1,)), ((), ())),
                                       preferred_element_type=_f32)


_ffn = pl.pallas_call(
    _ffn_body,
    grid_spec=pltpu.PrefetchScalarGridSpec(
        num_scalar_prefetch=2,
        grid=(NT,),
        in_specs=[
            pl.BlockSpec((BLK, D), lambda i, te, tv: (i, 0)),
            pl.BlockSpec((1, F, D), lambda i, te, tv: (te[i], 0, 0)),
            pl.BlockSpec((1, F, D), lambda i, te, tv: (te[i], 0, 0)),
            pl.BlockSpec((1, D, F), lambda i, te, tv: (te[i], 0, 0)),
        ],
        out_specs=pl.BlockSpec((BLK, D), lambda i, te, tv: (i, 0)),
    ),
    out_shape=jax.ShapeDtypeStruct((P_CAP, D), _f32),
)


def _combine_body(buf_hbm, pos0_hbm, pos1_hbm, w0_hbm, w1_hbm, out_hbm,
                  idx0_v, idx1_v, a_v, b_v, o_v, w0_v, w1_v, sem):
    wid = lax.axis_index("s") * NC + lax.axis_index("c")
    for c in range(TPW // KC):
        base = wid * TPW + c * KC
        pltpu.sync_copy(pos0_hbm.at[pl.ds(base, KC)], idx0_v)
        pltpu.sync_copy(pos1_hbm.at[pl.ds(base, KC)], idx1_v)
        pltpu.sync_copy(w0_hbm.at[pl.ds(base, KC)], w0_v)
        pltpu.sync_copy(w1_hbm.at[pl.ds(base, KC)], w1_v)
        pltpu.async_copy(buf_hbm.at[idx0_v], a_v, sem).wait()
        pltpu.async_copy(buf_hbm.at[idx1_v], b_v, sem).wait()

        def row(r, carry):
            wa = w0_v[r, pl.ds(0, 16)]
            wb = w1_v[r, pl.ds(0, 16)]

            def dchunk(j, inner):
                b0 = pl.multiple_of(j * 64, 64)
                for q in range(4):
                    sl = pl.ds(b0 + q * 16, 16)
                    o_v[r, sl] = wa * a_v[r, sl] + wb * b_v[r, sl]
                return inner

            lax.fori_loop(0, D // 64, dchunk, 0)
            return carry

        lax.fori_loop(0, KC, row, 0)
        pltpu.sync_copy(o_v, out_hbm.at[pl.ds(base, KC)])


@functools.cache
def _make_combine():
    return pl.kernel(
        _combine_body,
        out_type=jax.ShapeDtypeStruct((T, D), _f32),
        mesh=plsc.VectorSubcoreMesh(core_axis_name="c", subcore_axis_name="s",
                                    num_cores=NC, num_subcores=NS),
        scratch_types=[
            pltpu.VMEM((KC,), _i32),
            pltpu.VMEM((KC,), _i32),
            pltpu.VMEM((KC, D), _f32),
            pltpu.VMEM((KC, D), _f32),
            pltpu.VMEM((KC, D), _f32),
            pltpu.VMEM((KC, 128), _f32),
            pltpu.VMEM((KC, 128), _f32),
            pltpu.SemaphoreType.DMA,
        ],
    )


def kernel(hidden_states, gate_w, gate_proj_w, up_proj_w, down_proj_w):
    pos, w0b, w1b, te2, tv2 = _router(hidden_states, gate_w)
    pos_flat = pos.reshape(K * T)
    pos0, pos1 = pos_flat[:T], pos_flat[T:]
    te, tv = te2[0, :NT], tv2[0, :NT]
    xs = _make_dispatch()(hidden_states, pos0, pos1)
    buf = _ffn(te, tv, xs, gate_proj_w, up_proj_w, down_proj_w)
    return _make_combine()(buf, pos0, pos1, w0b, w1b)
